# baseline (speedup 1.0000x reference)
import functools

import jax
import jax.numpy as jnp
import numpy as np
from jax import lax
from jax.experimental import pallas as pl
from jax.experimental.pallas import tpu as pltpu
from jax.experimental.pallas import tpu_sc as plsc

_M = 512
_PAIRS = _M * (_M - 1) // 2
_RPC = 511
_BLK = 8 * _RPC
_NG = 16
_NC = 4


def _merge_matrices() -> tuple[np.ndarray, np.ndarray]:
    la = np.zeros((512, 1022), dtype=np.float32)
    lb = np.zeros((512, 1022), dtype=np.float32)
    r = np.arange(511)
    la[r, 2 * r] = 1.0
    lb[r, 2 * r + 1] = 1.0
    return la, lb


def _pack_matrix() -> np.ndarray:
    p = np.zeros((128, 64), dtype=np.float32)
    u = np.arange(64)
    p[2 * u, u] = 1.0
    p[2 * u + 1, u] = 65536.0
    return p


def _start_copies(a_hbm, b_hbm, bufs, sems, step, slot):
    for ch in range(2):
        pltpu.make_async_copy(
            a_hbm.at[pl.ds(step * 8, 8), :, ch, 0, :],
            bufs[ch].at[slot], sems.at[slot, ch]).start()
        pltpu.make_async_copy(
            b_hbm.at[pl.ds(step * 8, 8), :, ch, 0, :],
            bufs[2 + ch].at[slot], sems.at[slot, 2 + ch]).start()


def _wait_copies(a_hbm, b_hbm, bufs, sems, step, slot):
    for ch in range(2):
        pltpu.make_async_copy(
            a_hbm.at[pl.ds(step * 8, 8), :, ch, 0, :],
            bufs[ch].at[slot], sems.at[slot, ch]).wait()
        pltpu.make_async_copy(
            b_hbm.at[pl.ds(step * 8, 8), :, ch, 0, :],
            bufs[2 + ch].at[slot], sems.at[slot, 2 + ch]).wait()


def _phase1_body(a_hbm, b_hbm, la_ref, lb_ref, p_ref, o_ref,
                 a0b, a1b, b0b, b1b, sems, *, base, nsteps):
    b = pl.program_id(0) + base
    slot = lax.rem(b, 2)
    bufs = (a0b, a1b, b0b, b1b)

    @pl.when(b == base)
    def _():
        _start_copies(a_hbm, b_hbm, bufs, sems, base, lax.rem(base, 2))

    @pl.when(b < base + nsteps - 1)
    def _():
        _start_copies(a_hbm, b_hbm, bufs, sems, b + 1, 1 - slot)

    _wait_copies(a_hbm, b_hbm, bufs, sems, b, slot)

    s0 = a0b[slot] + b0b[slot]
    s1 = a1b[slot] + b1b[slot]
    x = (s0 >= s1).astype(jnp.bfloat16)
    for gamma in range(2):
        g4 = 4 * gamma
        xs = x[g4] + x[g4 + 1] + x[g4 + 2] + x[g4 + 3]
        ya = jnp.dot(la_ref[...], xs, preferred_element_type=jnp.float32)
        yb = jnp.dot(lb_ref[...], xs, preferred_element_type=jnp.float32)
        pa = jnp.dot(ya.astype(jnp.bfloat16), p_ref[...],
                     preferred_element_type=jnp.float32)
        pb = jnp.dot(yb.astype(jnp.bfloat16), p_ref[...],
                     preferred_element_type=jnp.float32)
        packed = jnp.concatenate([pa, pb], axis=1).astype(jnp.int32)
        o_ref[pl.ds(gamma * 512, 512), :] = packed


def _phase1(a5, b5, la, lb, pmat, base, nsteps):
    return pl.pallas_call(
        functools.partial(_phase1_body, base=base, nsteps=nsteps),
        grid=(nsteps,),
        in_specs=[
            pl.BlockSpec(memory_space=pl.ANY),
            pl.BlockSpec(memory_space=pl.ANY),
            pl.BlockSpec((512, 1022), lambda b: (0, 0)),
            pl.BlockSpec((512, 1022), lambda b: (0, 0)),
            pl.BlockSpec((128, 64), lambda b: (0, 0)),
        ],
        out_specs=pl.BlockSpec((1024, 128), lambda b: (b, 0)),
        out_shape=jax.ShapeDtypeStruct((nsteps * 1024, 128), jnp.int32),
        scratch_shapes=[
            pltpu.VMEM((2, 8, 1022, 128), jnp.float32),
            pltpu.VMEM((2, 8, 1022, 128), jnp.float32),
            pltpu.VMEM((2, 8, 1022, 128), jnp.float32),
            pltpu.VMEM((2, 8, 1022, 128), jnp.float32),
            pltpu.SemaphoreType.DMA((2, 4)),
        ],
    )(a5, b5, la, lb, pmat)


_BATCH = 8


@functools.cache
def _sc_expand_fn(copy_prev: bool):
    mesh = plsc.VectorSubcoreMesh(core_axis_name="c", subcore_axis_name="s")
    ng_out = _NG if copy_prev else _NG // 2
    body = functools.partial(_sc_expand, copy_prev=copy_prev)
    scratch = [
        pltpu.VMEM((512, 128), jnp.int32),
        pltpu.VMEM((2, _BATCH, _M), jnp.float32),
        pltpu.VMEM((_BATCH, _M), jnp.float32),
        pltpu.SemaphoreType.DMA,
    ]
    if copy_prev:
        scratch.append(pltpu.SemaphoreType.DMA)
    return pl.kernel(
        body,
        mesh=mesh,
        out_type=jax.ShapeDtypeStruct((ng_out, _M, _M), jnp.float32),
        scratch_types=scratch,
        compiler_params=pltpu.CompilerParams(needs_layout_passes=False),
    )


def _sc_expand(xsp_hbm, *args, copy_prev):
    if copy_prev:
        o1_hbm, out_hbm, plane_v, rows_v, zrow_v, dsem, csem = args
    else:
        out_hbm, plane_v, rows_v, zrow_v, dsem = args
    s_idx = lax.axis_index("s")
    c_idx = lax.axis_index("c")
    g_loc = s_idx >> 1
    quarter = (s_idx & 1) * 2 + c_idx
    g_out = g_loc + (8 if copy_prev else 0)
    nn = 512 - 32 * g_loc
    base_row = quarter * 128
    if copy_prev:
        prev_cp = pltpu.make_async_copy(
            o1_hbm.at[g_loc, pl.ds(base_row, 128)],
            out_hbm.at[g_loc, pl.ds(base_row, 128)], csem)
        prev_cp.start()
    pltpu.sync_copy(xsp_hbm.at[g_loc], plane_v)
    jot = lax.iota(jnp.int32, 16)
    zero16 = jnp.zeros((16,), jnp.float32)
    nb = 128 // _BATCH

    def zinit(k, _):
        for r in range(_BATCH):
            zrow_v[r, pl.ds(k * 16, 16)] = zero16
        return 0

    lax.fori_loop(0, 32, zinit, 0)
    vb = jnp.clip((nn - base_row + _BATCH - 1) // _BATCH, 0, nb)

    def _drain():
        return pltpu.make_async_copy(
            rows_v.at[0], out_hbm.at[g_out, pl.ds(base_row, _BATCH)], dsem)

    def batch_body(bidx, _):
        i0 = base_row + bidx * _BATCH
        slot = lax.rem(bidx, 2)
        valid = i0 < nn

        @pl.when(valid)
        def _():
            @pl.when(bidx >= 2)
            def _():
                _drain().wait()

            ucs = []
            for r in range(_BATCH):
                i = i0 + r
                ucs.append(i * 511 - ((i * (i - 1)) >> 1) - i - 1)

            def kloop(k, _):
                j = jot + k * 16
                lowb = j * 511 - ((j * (j - 1)) >> 1) - j - 1
                mcol = jnp.where(j < nn, 1.0, 0.0)
                for r in range(_BATCH):
                    i = i0 + r
                    idx = jnp.where(j > i, ucs[r] + j, lowb + i)
                    idx = jnp.where(j == i, 131071, idx)
                    w_idx = lax.shift_right_logical(idx, 1)
                    w = plsc.load_gather(
                        plane_v,
                        [lax.shift_right_logical(w_idx, 7), w_idx & 127])
                    v = lax.shift_right_logical(w, (idx & 1) << 4) & 0xFFFF
                    rows_v[slot, r, pl.ds(k * 16, 16)] = (
                        v.astype(jnp.float32) * mcol)
                return 0

            lax.fori_loop(0, 32, kloop, 0)
            pltpu.make_async_copy(
                rows_v.at[slot], out_hbm.at[g_out, pl.ds(i0, _BATCH)],
                dsem).start()

        @pl.when(jnp.logical_not(valid))
        def _():
            pltpu.sync_copy(zrow_v, out_hbm.at[g_out, pl.ds(i0, _BATCH)])

        return 0

    lax.fori_loop(0, nb, batch_body, 0)

    @pl.when(vb >= 1)
    def _():
        _drain().wait()

    @pl.when(vb >= 2)
    def _():
        _drain().wait()

    if copy_prev:
        prev_cp.wait()


def _as_tiles(x):
    return (x.reshape(64, 1022, 128, 2)
             .transpose(0, 1, 3, 2)
             .reshape(64, 1022, 2, 1, 128))


def kernel(adj_logits, gumbel_noise):
    a5 = _as_tiles(adj_logits)
    b5 = _as_tiles(gumbel_noise)
    la, lb = _merge_matrices()
    la = jnp.asarray(la, dtype=jnp.bfloat16)
    lb = jnp.asarray(lb, dtype=jnp.bfloat16)
    pmat = jnp.asarray(_pack_matrix(), dtype=jnp.bfloat16)
    xsp1 = _phase1(a5, b5, la, lb, pmat, 0, 4)
    xsp2 = _phase1(a5, b5, la, lb, pmat, 4, 4)
    out1 = _sc_expand_fn(False)(xsp1.reshape(8, _M, 128))
    padded = jnp.pad(out1, ((0, 8), (0, 0), (0, 0)))
    out2 = _sc_expand_fn(False)(xsp2.reshape(8, _M, 128))
    return lax.dynamic_update_slice(padded, out2, (8, 0, 0))


# speedup vs baseline: 1.0462x; 1.0462x over previous
import functools

import jax
import jax.numpy as jnp
import numpy as np
from jax import lax
from jax.experimental import pallas as pl
from jax.experimental.pallas import tpu as pltpu
from jax.experimental.pallas import tpu_sc as plsc

_M = 512
_PAIRS = _M * (_M - 1) // 2
_RPC = 511
_BLK = 8 * _RPC
_NG = 16
_NC = 4


def _merge_matrices() -> tuple[np.ndarray, np.ndarray]:
    la = np.zeros((512, 1022), dtype=np.float32)
    lb = np.zeros((512, 1022), dtype=np.float32)
    r = np.arange(511)
    la[r, 2 * r] = 1.0
    lb[r, 2 * r + 1] = 1.0
    return la, lb


def _pack_matrix() -> np.ndarray:
    p = np.zeros((128, 64), dtype=np.float32)
    u = np.arange(64)
    p[2 * u, u] = 1.0
    p[2 * u + 1, u] = 65536.0
    return p


def _start_copies(a_hbm, b_hbm, bufs, sems, step, slot):
    for ch in range(2):
        pltpu.make_async_copy(
            a_hbm.at[pl.ds(step * 8, 8), :, ch, 0, :],
            bufs[ch].at[slot], sems.at[slot, ch]).start()
        pltpu.make_async_copy(
            b_hbm.at[pl.ds(step * 8, 8), :, ch, 0, :],
            bufs[2 + ch].at[slot], sems.at[slot, 2 + ch]).start()


def _wait_copies(a_hbm, b_hbm, bufs, sems, step, slot):
    for ch in range(2):
        pltpu.make_async_copy(
            a_hbm.at[pl.ds(step * 8, 8), :, ch, 0, :],
            bufs[ch].at[slot], sems.at[slot, ch]).wait()
        pltpu.make_async_copy(
            b_hbm.at[pl.ds(step * 8, 8), :, ch, 0, :],
            bufs[2 + ch].at[slot], sems.at[slot, 2 + ch]).wait()


def _phase1_body(a_hbm, b_hbm, la_ref, lb_ref, p_ref, o_ref,
                 a0b, a1b, b0b, b1b, sems, *, base, nsteps):
    b = pl.program_id(0) + base
    slot = lax.rem(b, 2)
    bufs = (a0b, a1b, b0b, b1b)

    @pl.when(b == base)
    def _():
        _start_copies(a_hbm, b_hbm, bufs, sems, base, lax.rem(base, 2))

    @pl.when(b < base + nsteps - 1)
    def _():
        _start_copies(a_hbm, b_hbm, bufs, sems, b + 1, 1 - slot)

    _wait_copies(a_hbm, b_hbm, bufs, sems, b, slot)

    s0 = a0b[slot] + b0b[slot]
    s1 = a1b[slot] + b1b[slot]
    x = (s0 >= s1).astype(jnp.bfloat16)
    for gamma in range(2):
        g4 = 4 * gamma
        xs = x[g4] + x[g4 + 1] + x[g4 + 2] + x[g4 + 3]
        ya = jnp.dot(la_ref[...], xs, preferred_element_type=jnp.float32)
        yb = jnp.dot(lb_ref[...], xs, preferred_element_type=jnp.float32)
        pa = jnp.dot(ya.astype(jnp.bfloat16), p_ref[...],
                     preferred_element_type=jnp.float32)
        pb = jnp.dot(yb.astype(jnp.bfloat16), p_ref[...],
                     preferred_element_type=jnp.float32)
        packed = jnp.concatenate([pa, pb], axis=1).astype(jnp.int32)
        o_ref[pl.ds(gamma * 512, 512), :] = packed


def _phase1(a5, b5, la, lb, pmat, base, nsteps):
    return pl.pallas_call(
        functools.partial(_phase1_body, base=base, nsteps=nsteps),
        grid=(nsteps,),
        in_specs=[
            pl.BlockSpec(memory_space=pl.ANY),
            pl.BlockSpec(memory_space=pl.ANY),
            pl.BlockSpec((512, 1022), lambda b: (0, 0)),
            pl.BlockSpec((512, 1022), lambda b: (0, 0)),
            pl.BlockSpec((128, 64), lambda b: (0, 0)),
        ],
        out_specs=pl.BlockSpec((1024, 128), lambda b: (b, 0)),
        out_shape=jax.ShapeDtypeStruct((nsteps * 1024, 128), jnp.int32),
        scratch_shapes=[
            pltpu.VMEM((2, 8, 1022, 128), jnp.float32),
            pltpu.VMEM((2, 8, 1022, 128), jnp.float32),
            pltpu.VMEM((2, 8, 1022, 128), jnp.float32),
            pltpu.VMEM((2, 8, 1022, 128), jnp.float32),
            pltpu.SemaphoreType.DMA((2, 4)),
        ],
    )(a5, b5, la, lb, pmat)


_BATCH = 8


@functools.cache
def _sc_expand_fn():
    mesh = plsc.VectorSubcoreMesh(core_axis_name="c", subcore_axis_name="s")
    return pl.kernel(
        _sc_expand,
        mesh=mesh,
        out_type=jax.ShapeDtypeStruct((_NG, _M, _M), jnp.float32),
        scratch_types=[
            pltpu.VMEM((512, 128), jnp.int32),
            pltpu.VMEM((2, _BATCH, _M), jnp.float32),
            pltpu.VMEM((_BATCH, _M), jnp.float32),
            pltpu.SemaphoreType.DMA,
        ],
        compiler_params=pltpu.CompilerParams(needs_layout_passes=False),
    )


def _sc_expand(xsp_hbm, out_hbm, plane_v, rows_v, zrow_v, dsem):
    g_out = lax.axis_index("s")
    h = lax.axis_index("c")
    nn = 512 - 32 * lax.rem(g_out, 8)
    base_row = h * 256
    pltpu.sync_copy(xsp_hbm.at[g_out], plane_v)
    jot = lax.iota(jnp.int32, 16)
    zero16 = jnp.zeros((16,), jnp.float32)
    nb = 256 // _BATCH

    def zinit(k, _):
        for r in range(_BATCH):
            zrow_v[r, pl.ds(k * 16, 16)] = zero16
        return 0

    lax.fori_loop(0, 32, zinit, 0)
    vb = jnp.clip((nn - base_row + _BATCH - 1) // _BATCH, 0, nb)

    def _drain():
        return pltpu.make_async_copy(
            rows_v.at[0], out_hbm.at[g_out, pl.ds(base_row, _BATCH)], dsem)

    def batch_body(bidx, _):
        i0 = base_row + bidx * _BATCH
        slot = lax.rem(bidx, 2)
        valid = i0 < nn

        @pl.when(valid)
        def _():
            @pl.when(bidx >= 2)
            def _():
                _drain().wait()

            ucs = []
            for r in range(_BATCH):
                i = i0 + r
                ucs.append(i * 511 - ((i * (i - 1)) >> 1) - i - 1)

            def kloop(k, _):
                j = jot + k * 16
                lowb = j * 511 - ((j * (j - 1)) >> 1) - j - 1
                mcol = jnp.where(j < nn, 1.0, 0.0)
                for r in range(_BATCH):
                    i = i0 + r
                    idx = jnp.where(j > i, ucs[r] + j, lowb + i)
                    idx = jnp.where(j == i, 131071, idx)
                    w_idx = lax.shift_right_logical(idx, 1)
                    w = plsc.load_gather(
                        plane_v,
                        [lax.shift_right_logical(w_idx, 7), w_idx & 127])
                    v = lax.shift_right_logical(w, (idx & 1) << 4) & 0xFFFF
                    rows_v[slot, r, pl.ds(k * 16, 16)] = (
                        v.astype(jnp.float32) * mcol)
                return 0

            lax.fori_loop(0, 32, kloop, 0)
            pltpu.make_async_copy(
                rows_v.at[slot], out_hbm.at[g_out, pl.ds(i0, _BATCH)],
                dsem).start()

        @pl.when(jnp.logical_not(valid))
        def _():
            pltpu.sync_copy(zrow_v, out_hbm.at[g_out, pl.ds(i0, _BATCH)])

        return 0

    lax.fori_loop(0, nb, batch_body, 0)

    @pl.when(vb >= 1)
    def _():
        _drain().wait()

    @pl.when(vb >= 2)
    def _():
        _drain().wait()


def _as_tiles(x):
    return (x.reshape(64, 1022, 128, 2)
             .transpose(0, 1, 3, 2)
             .reshape(64, 1022, 2, 1, 128))


def kernel(adj_logits, gumbel_noise):
    a5 = _as_tiles(adj_logits)
    b5 = _as_tiles(gumbel_noise)
    la, lb = _merge_matrices()
    la = jnp.asarray(la, dtype=jnp.bfloat16)
    lb = jnp.asarray(lb, dtype=jnp.bfloat16)
    pmat = jnp.asarray(_pack_matrix(), dtype=jnp.bfloat16)
    xsp = _phase1(a5, b5, la, lb, pmat, 0, 8)
    return _sc_expand_fn()(xsp.reshape(_NG, _M, 128))
